# baseline (speedup 1.0000x reference)
import functools

import jax
import jax.numpy as jnp
from jax.experimental import pallas as pl
from jax.experimental.pallas import tpu as pltpu

K_TOP = 971
N_NODES = 10000
EMB = 128


def _score_body(x_ref, w1_ref, b1_ref, w2_ref, b2_ref, s_ref):
    xb = x_ref[...]
    h0 = jnp.dot(xb, w1_ref[...], preferred_element_type=jnp.float32)
    h0 = jnp.maximum(h0 + b1_ref[...][None, :], 0.0)
    logit = jnp.dot(h0, w2_ref[...], preferred_element_type=jnp.float32)
    logit = logit + b2_ref[...][None, :]
    s_ref[...] = jax.nn.sigmoid(logit)


def _scores_pallas(x, W1, b1, W2, b2):
    BR = 400
    grid = (N_NODES // BR,)
    return pl.pallas_call(
        _score_body,
        grid=grid,
        in_specs=[
            pl.BlockSpec((BR, EMB), lambda i: (i, 0)),
            pl.BlockSpec((EMB, EMB), lambda i: (0, 0)),
            pl.BlockSpec((EMB,), lambda i: (0,)),
            pl.BlockSpec((EMB, 1), lambda i: (0, 0)),
            pl.BlockSpec((1,), lambda i: (0,)),
        ],
        out_specs=pl.BlockSpec((BR, 1), lambda i: (i, 0)),
        out_shape=jax.ShapeDtypeStruct((N_NODES, 1), jnp.float32),
    )(x, W1, b1, W2, b2)


def kernel(x, edge_index, edge_attr, edge_type, batch, nindex,
           W1, b1, W2, b2, Wg, bg, mask_emb, mask_attr, mask_type):
    scores = _scores_pallas(x, W1, b1, W2, b2)[:, 0]
    idx1 = jnp.argsort(-scores)
    idx = idx1[:K_TOP]
    x_masked = x.at[idx].set(mask_emb)

    def gnn(feat):
        src = edge_index[0]
        dst = edge_index[1]
        msgs = feat[src]
        agg = jax.ops.segment_sum(msgs, dst, num_segments=feat.shape[0])
        return jax.nn.relu(agg @ Wg + bg)

    h = gnn(x_masked)
    edge_attr_new = edge_attr.at[nindex].set(mask_attr)
    edge_type_new = edge_type.at[nindex].set(mask_type)
    hnew = gnn(h)
    return (hnew, h, idx, edge_attr_new, edge_type_new)


# speedup vs baseline: 3.9131x; 3.9131x over previous
import functools

import jax
import jax.numpy as jnp
from jax import lax
from jax.experimental import pallas as pl
from jax.experimental.pallas import tpu as pltpu
from jax.experimental.pallas import tpu_sc as plsc

K_TOP = 971
N = 10000
EMB = 128
NB = 79
NPAD = NB * 128

NE = 320000
E_ROWS_W = 80
E_ROWS = 32 * E_ROWS_W
NE_PAD = E_ROWS * 128
ACC_ROWS = 10240
ACC_ZROWS = ACC_ROWS // 16

NI = 224000
NI_ROWS_W = 112
NI_ROWS = 16 * NI_ROWS_W
NI_PAD = NI_ROWS * 128
FLAG_LEN = 321536
FLAG_ZLEN = FLAG_LEN // 16
E_W = NE // 32
E_CHUNK = 2000


def _topk_body(xp_ref, w1_ref, b1_ref, w2_ref, b2_ref, me_ref,
               xm_ref, idx_ref, srow_ref):
    f32 = jnp.float32
    w1 = w1_ref[...]
    b1 = b1_ref[...]
    w2 = w2_ref[...]
    b2 = b2_ref[...]
    eye = (lax.broadcasted_iota(jnp.int32, (128, 128), 0)
           == lax.broadcasted_iota(jnp.int32, (128, 128), 1)).astype(f32)
    lane_row = lax.broadcasted_iota(jnp.int32, (1, 128), 1)

    def score_blk(i, scol):
        xb = xp_ref[pl.ds(i * 128, 128), :]
        h0 = jnp.maximum(
            jnp.dot(xb, w1, preferred_element_type=f32) + b1[None, :], 0.0)
        logit = jnp.dot(h0, w2, preferred_element_type=f32) + b2
        s = jax.nn.sigmoid(logit)
        gid = i * 128 + lax.broadcasted_iota(jnp.int32, (128, 1), 0)
        s = jnp.where(gid < N, s, 0.0)
        e_i = (lane_row == i).astype(f32)
        return scol + jnp.dot(s, e_i, preferred_element_type=f32,
                              precision=lax.Precision.HIGHEST)

    scol = lax.fori_loop(0, NB, score_blk, jnp.zeros((128, 128), f32))
    srow_ref[...] = lax.dot_general(
        scol, eye, (((0,), (0,)), ((), ())), preferred_element_type=f32,
        precision=lax.Precision.HIGHEST)

    ones_col = jnp.ones((128, 1), f32)
    me_row = me_ref[...][None, :]
    tgt = lax.broadcasted_iota(jnp.int32, (1, 1024), 1).astype(f32)

    def rank_i(i, idx_acc):
        row_i = srow_ref[pl.ds(i, 1), :]
        si = lax.dot_general(
            eye, row_i, (((1,), (1,)), ((), ())), preferred_element_type=f32,
            precision=lax.Precision.HIGHEST)
        gi = i * 128 + lax.broadcasted_iota(jnp.int32, (128, 1), 0)

        def rank_j(j, acc):
            sj = srow_ref[pl.ds(j, 1), :]
            gj = j * 128 + lane_row
            cmp = jnp.where((sj > si) | ((sj == si) & (gj < gi)), 1.0, 0.0)
            return acc + jnp.dot(cmp, ones_col, preferred_element_type=f32)

        acc = lax.fori_loop(0, NB, rank_j, jnp.zeros((128, 1), f32))
        mask = acc < float(K_TOP)
        xb = xp_ref[pl.ds(i * 128, 128), :]
        xm_ref[pl.ds(i * 128, 128), :] = jnp.where(mask, me_row, xb)
        m = jnp.where(acc == tgt, 1.0, 0.0)
        return idx_acc + jnp.sum(m * gi.astype(f32), axis=0, keepdims=True)

    idxf = lax.fori_loop(0, NB, rank_i, jnp.zeros((1, 1024), f32))
    idx_ref[...] = idxf.astype(jnp.int32)


def _topk_mask(xp, W1, b1, W2, b2, mask_emb):
    return pl.pallas_call(
        _topk_body,
        out_shape=(jax.ShapeDtypeStruct((NPAD, EMB), jnp.float32),
                   jax.ShapeDtypeStruct((1, 1024), jnp.int32)),
        scratch_shapes=[pltpu.VMEM((128, 128), jnp.float32)],
    )(xp, W1, b1, W2, b2, mask_emb)


def _lin_body(p_ref, wg_ref, bg_ref, o_ref):
    a = p_ref[0, :, :] + p_ref[1, :, :]
    h = jnp.dot(a, wg_ref[...], preferred_element_type=jnp.float32) + bg_ref[...][None, :]
    o_ref[...] = jnp.maximum(h, 0.0)


def _linear(parts, Wg, bg):
    BR = 400
    return pl.pallas_call(
        _lin_body,
        grid=(N // BR,),
        in_specs=[pl.BlockSpec((2, BR, EMB), lambda i: (0, i, 0)),
                  pl.BlockSpec((EMB, EMB), lambda i: (0, 0)),
                  pl.BlockSpec((EMB,), lambda i: (0,))],
        out_specs=pl.BlockSpec((BR, EMB), lambda i: (i, 0)),
        out_shape=jax.ShapeDtypeStruct((N, EMB), jnp.float32),
    )(parts, Wg, bg)


@functools.cache
def _sc_mesh():
    return plsc.VectorSubcoreMesh(core_axis_name="c", subcore_axis_name="s",
                                  num_cores=2, num_subcores=16)


def _seg_sum(feat, src, dst):
    call = pl.kernel(
        _seg_sum_body,
        out_type=jax.ShapeDtypeStruct((2, ACC_ROWS, EMB), jnp.float32),
        mesh=_sc_mesh(),
        scratch_types=[
            pltpu.VMEM((E_ROWS_W // 2, 128), jnp.int32),
            pltpu.VMEM((E_ROWS_W // 2, 128), jnp.int32),
            pltpu.VMEM((128, EMB), jnp.float32),
            pltpu.VMEM((128, EMB), jnp.float32),
            pltpu.VMEM_SHARED((ACC_ROWS, EMB), jnp.float32),
            pltpu.SemaphoreType.DMA,
            pltpu.SemaphoreType.DMA,
        ],
    )
    return call(feat, src, dst)


def _seg_sum_body(feat_hbm, src_hbm, dst_hbm, out_hbm,
                  srcv, dstv, rows_a, rows_b, acc_sh, sem_a, sem_b):
    c = lax.axis_index("c")
    s = lax.axis_index("s")
    wid = s * 2 + c
    half_rows = E_ROWS_W // 2

    def zfill(i, _):
        rows_a[i // 8, pl.ds((i % 8) * 16, 16)] = jnp.zeros((16,), jnp.float32)
        return 0

    lax.fori_loop(0, 128 * 8, zfill, 0)
    zbase = s * ACC_ZROWS
    for k in range(ACC_ZROWS // 128):
        pltpu.sync_copy(rows_a, acc_sh.at[pl.ds(zbase + k * 128, 128)])
    plsc.subcore_barrier()

    for half in range(2):
        row0 = wid * E_ROWS_W + half * half_rows
        pltpu.sync_copy(src_hbm.at[pl.ds(row0, half_rows)], srcv)
        pltpu.sync_copy(dst_hbm.at[pl.ds(row0, half_rows)], dstv)
        pltpu.async_copy(feat_hbm.at[srcv.at[0]], rows_a, sem_a)

        def pair(p, _):
            j = p * 2
            pltpu.async_copy(feat_hbm.at[srcv.at[j + 1]], rows_b, sem_b)
            pltpu.make_async_copy(feat_hbm.at[srcv.at[j]], rows_a, sem_a).wait()
            pltpu.sync_copy(rows_a, acc_sh.at[dstv.at[j]], add=True)

            @pl.when(j + 2 < half_rows)
            def _start_next():
                pltpu.async_copy(feat_hbm.at[srcv.at[j + 2]], rows_a, sem_a)

            pltpu.make_async_copy(feat_hbm.at[srcv.at[j + 1]], rows_b, sem_b).wait()
            pltpu.sync_copy(rows_b, acc_sh.at[dstv.at[j + 1]], add=True)
            return 0

        lax.fori_loop(0, half_rows // 2, pair, 0)

    plsc.subcore_barrier()
    obase = s * ACC_ZROWS
    pltpu.sync_copy(acc_sh.at[pl.ds(obase, ACC_ZROWS)],
                    out_hbm.at[c, pl.ds(obase, ACC_ZROWS)])


def _edge_mask(ea, et, nidx, ma, mt):
    call = pl.kernel(
        _edge_mask_body,
        out_type=(jax.ShapeDtypeStruct((NE * 4,), jnp.float32),
                  jax.ShapeDtypeStruct((NE,), jnp.float32)),
        mesh=_sc_mesh(),
        scratch_types=[
            pltpu.VMEM((NI_ROWS_W, 128), jnp.int32),
            pltpu.VMEM((128,), jnp.int32),
            pltpu.VMEM((FLAG_ZLEN // 8,), jnp.int32),
            pltpu.VMEM((E_CHUNK,), jnp.int32),
            pltpu.VMEM((E_CHUNK * 4,), jnp.float32),
            pltpu.VMEM((E_CHUNK * 4,), jnp.float32),
            pltpu.VMEM((E_CHUNK,), jnp.float32),
            pltpu.VMEM((E_CHUNK,), jnp.float32),
            pltpu.VMEM((16,), jnp.float32),
            pltpu.VMEM((16,), jnp.float32),
            pltpu.VMEM_SHARED((FLAG_LEN,), jnp.int32),
            pltpu.SemaphoreType.DMA,
        ],
    )
    return call(ea, et, nidx, ma, mt)


def _edge_mask_body(ea_hbm, et_hbm, nidx_hbm, ma_hbm, mt_hbm, oa_hbm, ot_hbm,
               idxv, onesv, zerov, flagv, eav, oav, etv, otv, mav, mtv,
               flag_sh, sem):
    c = lax.axis_index("c")
    s = lax.axis_index("s")
    wid = s * 2 + c

    zlen = FLAG_ZLEN // 8

    def zfill(i, _):
        zerov[pl.ds(i * 16, 16)] = jnp.zeros((16,), jnp.int32)
        return 0

    lax.fori_loop(0, zlen // 16, zfill, 0)

    def ofill(i, _):
        onesv[pl.ds(i * 16, 16)] = jnp.ones((16,), jnp.int32)
        return 0

    lax.fori_loop(0, 8, ofill, 0)

    fbase = s * FLAG_ZLEN
    for k in range(8):
        pltpu.sync_copy(zerov, flag_sh.at[pl.ds(fbase + k * zlen, zlen)])
    plsc.subcore_barrier()

    pltpu.sync_copy(nidx_hbm.at[pl.ds(s * NI_ROWS_W, NI_ROWS_W)], idxv)

    def scat(r, _):
        pltpu.async_copy(onesv, flag_sh.at[idxv.at[r]], sem)
        return 0

    lax.fori_loop(0, NI_ROWS_W, scat, 0)

    def drain(r, _):
        pltpu.make_async_copy(onesv, flag_sh.at[idxv.at[0]], sem).wait()
        return 0

    lax.fori_loop(0, NI_ROWS_W, drain, 0)
    plsc.subcore_barrier()

    pltpu.sync_copy(ma_hbm, mav)
    pltpu.sync_copy(mt_hbm, mtv)
    ma_vec = mav[...]
    mt_vec = mtv[...]
    lanes = lax.broadcasted_iota(jnp.int32, (16,), 0)
    rep_idx = [lax.shift_right_logical(lanes, 2) + 4 * b for b in range(4)]
    gdn = lax.GatherDimensionNumbers(
        offset_dims=(), collapsed_slice_dims=(0,), start_index_map=(0,))

    ebase = wid * E_W

    def chunk(q, _):
        e0 = ebase + q * E_CHUNK
        pltpu.sync_copy(flag_sh.at[pl.ds(e0, E_CHUNK)], flagv)
        pltpu.sync_copy(ea_hbm.at[pl.ds(e0 * 4, E_CHUNK * 4)], eav)
        pltpu.sync_copy(et_hbm.at[pl.ds(e0, E_CHUNK)], etv)

        def grp(g, _):
            fo = g * 16
            f = flagv[pl.ds(fo, 16)]
            ev = etv[pl.ds(fo, 16)]
            otv[pl.ds(fo, 16)] = jnp.where(f > 0, mt_vec, ev)
            for b in range(4):
                fexp = lax.gather(
                    f, rep_idx[b][:, None], gdn, (1,),
                    mode=lax.GatherScatterMode.PROMISE_IN_BOUNDS)
                o = fo * 4 + b * 16
                v = eav[pl.ds(o, 16)]
                oav[pl.ds(o, 16)] = jnp.where(fexp > 0, ma_vec, v)
            return 0

        lax.fori_loop(0, E_CHUNK // 16, grp, 0)

        pltpu.sync_copy(oav, oa_hbm.at[pl.ds(e0 * 4, E_CHUNK * 4)])
        pltpu.sync_copy(otv, ot_hbm.at[pl.ds(e0, E_CHUNK)])
        return 0

    lax.fori_loop(0, E_W // E_CHUNK, chunk, 0)


def kernel(x, edge_index, edge_attr, edge_type, batch, nindex,
           W1, b1, W2, b2, Wg, bg, mask_emb, mask_attr, mask_type):
    f32 = jnp.float32
    xp = jnp.pad(x, ((0, NPAD - N), (0, 0)))
    xm_tab, idx2d = _topk_mask(xp, W1, b1, W2, b2, mask_emb)
    idx = idx2d.reshape(1024)[:K_TOP]

    pad_e = NE_PAD - NE
    pad_i = jnp.arange(pad_e, dtype=jnp.int32)
    srcp = jnp.concatenate([edge_index[0], pad_i % N]).reshape(E_ROWS, 128)
    dstp = jnp.concatenate([edge_index[1], N + pad_i % (ACC_ROWS - N)]
                           ).reshape(E_ROWS, 128)

    p1 = _seg_sum(xm_tab, srcp, dstp)
    h = _linear(p1, Wg, bg)
    p2 = _seg_sum(h, srcp, dstp)
    hnew = _linear(p2, Wg, bg)

    pad_n = NI_PAD - NI
    nidxp = jnp.concatenate(
        [nindex.astype(jnp.int32),
         NE + jnp.arange(pad_n, dtype=jnp.int32) % (FLAG_LEN - NE)]
    ).reshape(NI_ROWS, 128)
    oa, ot = _edge_mask(edge_attr.reshape(NE * 4), edge_type.reshape(NE),
                        nidxp, jnp.tile(mask_attr, 4).astype(f32),
                        jnp.tile(mask_type, 16).astype(f32))
    edge_attr_new = oa.reshape(NE, 4)
    edge_type_new = ot.reshape(NE, 1)
    return (hnew, h, idx, edge_attr_new, edge_type_new)


# speedup vs baseline: 5.9254x; 1.5142x over previous
import functools

import jax
import jax.numpy as jnp
from jax import lax
from jax.experimental import pallas as pl
from jax.experimental.pallas import tpu as pltpu
from jax.experimental.pallas import tpu_sc as plsc

K_TOP = 971
N = 10000
EMB = 128
NB = 79
NPAD = NB * 128

NE = 320000
E_ROWS_W = 80
E_ROWS = 32 * E_ROWS_W
NE_PAD = E_ROWS * 128
ACC_ROWS = 10240
ACC_ZROWS = ACC_ROWS // 16

NI = 224000
NI_ROWS_W = 112
NI_ROWS = 16 * NI_ROWS_W
NI_PAD = NI_ROWS * 128
FLAG_LEN = 321536
FLAG_ZLEN = FLAG_LEN // 16
E_W = NE // 32
E_CHUNK = 2000


def _topk_body(xp_ref, w1_ref, b1_ref, w2_ref, b2_ref, me_ref,
               xm_ref, idx_ref, srow_ref, swide_ref):
    f32 = jnp.float32
    w1 = w1_ref[...]
    b1 = b1_ref[...]
    w2 = w2_ref[...]
    b2 = b2_ref[...]
    eye = (lax.broadcasted_iota(jnp.int32, (128, 128), 0)
           == lax.broadcasted_iota(jnp.int32, (128, 128), 1)).astype(f32)
    sub128 = lax.broadcasted_iota(jnp.int32, (128, 1), 0)
    row16 = lax.broadcasted_iota(jnp.int32, (16, 1), 0)
    blk1024 = lax.broadcasted_iota(jnp.int32, (1, 1024), 1) // 128

    def score_blk(i, carry):
        srow2, swide2 = carry
        xb = xp_ref[pl.ds(i * 128, 128), :]
        h0 = jnp.maximum(
            jnp.dot(xb, w1, preferred_element_type=f32) + b1[None, :], 0.0)
        logit = jnp.dot(h0, w2, preferred_element_type=f32) + b2
        s = jax.nn.sigmoid(logit)
        gid = i * 128 + sub128
        s = jnp.where(gid < N, s, 0.0)
        s_row = lax.dot_general(
            s, eye, (((0,), (0,)), ((), ())), preferred_element_type=f32,
            precision=lax.Precision.HIGHEST)
        srow2 = jnp.where(sub128 == i, s_row, srow2)
        s_big = jnp.concatenate([s_row] * 8, axis=1)
        pos = (row16 == i // 8) & (blk1024 == i % 8)
        swide2 = jnp.where(pos, s_big, swide2)
        return srow2, swide2

    srow2, swide2 = lax.fori_loop(
        0, NB, score_blk,
        (jnp.zeros((128, 128), f32), jnp.zeros((16, 1024), f32)))
    srow_ref[...] = srow2
    swide_ref[...] = swide2

    ones_k = jnp.ones((1024, 1), f32)
    me_row = me_ref[...][None, :]
    tgt = lax.broadcasted_iota(jnp.int32, (1, 1024), 1).astype(f32)
    lane_k = lax.broadcasted_iota(jnp.int32, (1, 1024), 1)

    def rank_i(i, idx_acc):
        row_i = srow_ref[pl.ds(i, 1), :]
        si = lax.dot_general(
            eye, row_i, (((1,), (1,)), ((), ())), preferred_element_type=f32,
            precision=lax.Precision.HIGHEST)
        gi = i * 128 + sub128

        def rank_j(jj, acc):
            sj = swide_ref[pl.ds(jj, 1), :]
            gj = jj * 1024 + lane_k
            cmp = jnp.where((sj > si) | ((sj == si) & (gj < gi)), 1.0, 0.0)
            return acc + jnp.dot(cmp, ones_k, preferred_element_type=f32)

        acc = lax.fori_loop(0, 16, rank_j, jnp.zeros((128, 1), f32))
        mask = acc < float(K_TOP)
        xb = xp_ref[pl.ds(i * 128, 128), :]
        xm_ref[pl.ds(i * 128, 128), :] = jnp.where(mask, me_row, xb)
        m = jnp.where(acc == tgt, 1.0, 0.0)
        return idx_acc + jnp.sum(m * gi.astype(f32), axis=0, keepdims=True)

    idxf = lax.fori_loop(0, NB, rank_i, jnp.zeros((1, 1024), f32))
    idx_ref[...] = idxf.astype(jnp.int32)


def _topk_mask(xp, W1, b1, W2, b2, mask_emb):
    return pl.pallas_call(
        _topk_body,
        out_shape=(jax.ShapeDtypeStruct((NPAD, EMB), jnp.float32),
                   jax.ShapeDtypeStruct((1, 1024), jnp.int32)),
        scratch_shapes=[pltpu.VMEM((128, 128), jnp.float32),
                        pltpu.VMEM((16, 1024), jnp.float32)],
    )(xp, W1, b1, W2, b2, mask_emb)


def _lin_body(p_ref, wg_ref, bg_ref, o_ref):
    a = p_ref[0, :, :] + p_ref[1, :, :]
    h = jnp.dot(a, wg_ref[...], preferred_element_type=jnp.float32) + bg_ref[...][None, :]
    o_ref[...] = jnp.maximum(h, 0.0)


def _linear(parts, Wg, bg):
    BR = 400
    return pl.pallas_call(
        _lin_body,
        grid=(N // BR,),
        in_specs=[pl.BlockSpec((2, BR, EMB), lambda i: (0, i, 0)),
                  pl.BlockSpec((EMB, EMB), lambda i: (0, 0)),
                  pl.BlockSpec((EMB,), lambda i: (0,))],
        out_specs=pl.BlockSpec((BR, EMB), lambda i: (i, 0)),
        out_shape=jax.ShapeDtypeStruct((N, EMB), jnp.float32),
    )(parts, Wg, bg)


@functools.cache
def _sc_mesh():
    return plsc.VectorSubcoreMesh(core_axis_name="c", subcore_axis_name="s",
                                  num_cores=2, num_subcores=16)


def _seg_sum(feat, src, dst):
    call = pl.kernel(
        _seg_sum_body,
        out_type=jax.ShapeDtypeStruct((2, ACC_ROWS, EMB), jnp.float32),
        mesh=_sc_mesh(),
        scratch_types=[
            pltpu.VMEM((E_ROWS_W // 2, 128), jnp.int32),
            pltpu.VMEM((E_ROWS_W // 2, 128), jnp.int32),
            pltpu.VMEM((128, EMB), jnp.float32),
            pltpu.VMEM((128, EMB), jnp.float32),
            pltpu.VMEM_SHARED((ACC_ROWS, EMB), jnp.float32),
            pltpu.SemaphoreType.DMA,
            pltpu.SemaphoreType.DMA,
        ],
    )
    return call(feat, src, dst)


def _seg_sum_body(feat_hbm, src_hbm, dst_hbm, out_hbm,
                  srcv, dstv, rows_a, rows_b, acc_sh, sem_a, sem_b):
    c = lax.axis_index("c")
    s = lax.axis_index("s")
    wid = s * 2 + c
    half_rows = E_ROWS_W // 2

    def zfill(i, _):
        rows_a[i // 8, pl.ds((i % 8) * 16, 16)] = jnp.zeros((16,), jnp.float32)
        return 0

    lax.fori_loop(0, 128 * 8, zfill, 0)
    zbase = s * ACC_ZROWS
    for k in range(ACC_ZROWS // 128):
        pltpu.sync_copy(rows_a, acc_sh.at[pl.ds(zbase + k * 128, 128)])
    plsc.subcore_barrier()

    for half in range(2):
        row0 = wid * E_ROWS_W + half * half_rows
        pltpu.sync_copy(src_hbm.at[pl.ds(row0, half_rows)], srcv)
        pltpu.sync_copy(dst_hbm.at[pl.ds(row0, half_rows)], dstv)
        pltpu.async_copy(feat_hbm.at[srcv.at[0]], rows_a, sem_a)

        def pair(p, _):
            j = p * 2
            pltpu.async_copy(feat_hbm.at[srcv.at[j + 1]], rows_b, sem_b)
            pltpu.make_async_copy(feat_hbm.at[srcv.at[j]], rows_a, sem_a).wait()
            pltpu.sync_copy(rows_a, acc_sh.at[dstv.at[j]], add=True)

            @pl.when(j + 2 < half_rows)
            def _start_next():
                pltpu.async_copy(feat_hbm.at[srcv.at[j + 2]], rows_a, sem_a)

            pltpu.make_async_copy(feat_hbm.at[srcv.at[j + 1]], rows_b, sem_b).wait()
            pltpu.sync_copy(rows_b, acc_sh.at[dstv.at[j + 1]], add=True)
            return 0

        lax.fori_loop(0, half_rows // 2, pair, 0)

    plsc.subcore_barrier()
    obase = s * ACC_ZROWS
    pltpu.sync_copy(acc_sh.at[pl.ds(obase, ACC_ZROWS)],
                    out_hbm.at[c, pl.ds(obase, ACC_ZROWS)])


def _edge_mask(ea, et, nidx, ma, mt):
    call = pl.kernel(
        _edge_mask_body,
        out_type=(jax.ShapeDtypeStruct((NE * 4,), jnp.float32),
                  jax.ShapeDtypeStruct((NE,), jnp.float32)),
        mesh=_sc_mesh(),
        scratch_types=[
            pltpu.VMEM((NI_ROWS_W, 128), jnp.int32),
            pltpu.VMEM((128,), jnp.int32),
            pltpu.VMEM((FLAG_ZLEN // 8,), jnp.int32),
            pltpu.VMEM((E_CHUNK,), jnp.int32),
            pltpu.VMEM((E_CHUNK * 4,), jnp.float32),
            pltpu.VMEM((E_CHUNK * 4,), jnp.float32),
            pltpu.VMEM((E_CHUNK,), jnp.float32),
            pltpu.VMEM((E_CHUNK,), jnp.float32),
            pltpu.VMEM((16,), jnp.float32),
            pltpu.VMEM((16,), jnp.float32),
            pltpu.VMEM_SHARED((FLAG_LEN,), jnp.int32),
            pltpu.SemaphoreType.DMA,
        ],
    )
    return call(ea, et, nidx, ma, mt)


def _edge_mask_body(ea_hbm, et_hbm, nidx_hbm, ma_hbm, mt_hbm, oa_hbm, ot_hbm,
               idxv, onesv, zerov, flagv, eav, oav, etv, otv, mav, mtv,
               flag_sh, sem):
    c = lax.axis_index("c")
    s = lax.axis_index("s")
    wid = s * 2 + c

    zlen = FLAG_ZLEN // 8

    def zfill(i, _):
        zerov[pl.ds(i * 16, 16)] = jnp.zeros((16,), jnp.int32)
        return 0

    lax.fori_loop(0, zlen // 16, zfill, 0)

    def ofill(i, _):
        onesv[pl.ds(i * 16, 16)] = jnp.ones((16,), jnp.int32)
        return 0

    lax.fori_loop(0, 8, ofill, 0)

    fbase = s * FLAG_ZLEN
    for k in range(8):
        pltpu.sync_copy(zerov, flag_sh.at[pl.ds(fbase + k * zlen, zlen)])
    plsc.subcore_barrier()

    pltpu.sync_copy(nidx_hbm.at[pl.ds(s * NI_ROWS_W, NI_ROWS_W)], idxv)

    def scat(r, _):
        pltpu.async_copy(onesv, flag_sh.at[idxv.at[r]], sem)
        return 0

    lax.fori_loop(0, NI_ROWS_W, scat, 0)

    def drain(r, _):
        pltpu.make_async_copy(onesv, flag_sh.at[idxv.at[0]], sem).wait()
        return 0

    lax.fori_loop(0, NI_ROWS_W, drain, 0)
    plsc.subcore_barrier()

    pltpu.sync_copy(ma_hbm, mav)
    pltpu.sync_copy(mt_hbm, mtv)
    ma_vec = mav[...]
    mt_vec = mtv[...]
    lanes = lax.broadcasted_iota(jnp.int32, (16,), 0)
    rep_idx = [lax.shift_right_logical(lanes, 2) + 4 * b for b in range(4)]
    gdn = lax.GatherDimensionNumbers(
        offset_dims=(), collapsed_slice_dims=(0,), start_index_map=(0,))

    ebase = wid * E_W

    def chunk(q, _):
        e0 = ebase + q * E_CHUNK
        pltpu.sync_copy(flag_sh.at[pl.ds(e0, E_CHUNK)], flagv)
        pltpu.sync_copy(ea_hbm.at[pl.ds(e0 * 4, E_CHUNK * 4)], eav)
        pltpu.sync_copy(et_hbm.at[pl.ds(e0, E_CHUNK)], etv)

        def grp(g, _):
            fo = g * 16
            f = flagv[pl.ds(fo, 16)]
            ev = etv[pl.ds(fo, 16)]
            otv[pl.ds(fo, 16)] = jnp.where(f > 0, mt_vec, ev)
            for b in range(4):
                fexp = lax.gather(
                    f, rep_idx[b][:, None], gdn, (1,),
                    mode=lax.GatherScatterMode.PROMISE_IN_BOUNDS)
                o = fo * 4 + b * 16
                v = eav[pl.ds(o, 16)]
                oav[pl.ds(o, 16)] = jnp.where(fexp > 0, ma_vec, v)
            return 0

        lax.fori_loop(0, E_CHUNK // 16, grp, 0)

        pltpu.sync_copy(oav, oa_hbm.at[pl.ds(e0 * 4, E_CHUNK * 4)])
        pltpu.sync_copy(otv, ot_hbm.at[pl.ds(e0, E_CHUNK)])
        return 0

    lax.fori_loop(0, E_W // E_CHUNK, chunk, 0)


def kernel(x, edge_index, edge_attr, edge_type, batch, nindex,
           W1, b1, W2, b2, Wg, bg, mask_emb, mask_attr, mask_type):
    f32 = jnp.float32
    xp = jnp.pad(x, ((0, NPAD - N), (0, 0)))
    xm_tab, idx2d = _topk_mask(xp, W1, b1, W2, b2, mask_emb)
    idx = idx2d.reshape(1024)[:K_TOP]

    pad_e = NE_PAD - NE
    pad_i = jnp.arange(pad_e, dtype=jnp.int32)
    srcp = jnp.concatenate([edge_index[0], pad_i % N]).reshape(E_ROWS, 128)
    dstp = jnp.concatenate([edge_index[1], N + pad_i % (ACC_ROWS - N)]
                           ).reshape(E_ROWS, 128)

    p1 = _seg_sum(xm_tab, srcp, dstp)
    h = _linear(p1, Wg, bg)
    p2 = _seg_sum(h, srcp, dstp)
    hnew = _linear(p2, Wg, bg)

    pad_n = NI_PAD - NI
    nidxp = jnp.concatenate(
        [nindex.astype(jnp.int32),
         NE + jnp.arange(pad_n, dtype=jnp.int32) % (FLAG_LEN - NE)]
    ).reshape(NI_ROWS, 128)
    oa, ot = _edge_mask(edge_attr.reshape(NE * 4), edge_type.reshape(NE),
                        nidxp, jnp.tile(mask_attr, 4).astype(f32),
                        jnp.tile(mask_type, 16).astype(f32))
    edge_attr_new = oa.reshape(NE, 4)
    edge_type_new = ot.reshape(NE, 1)
    return (hnew, h, idx, edge_attr_new, edge_type_new)
